# baseline (speedup 1.0000x reference)
import functools
import numpy as np
import jax
import jax.numpy as jnp
from jax import lax
from jax.experimental import pallas as pl
from jax.experimental.pallas import tpu as pltpu
from jax.experimental.pallas import tpu_sc as plsc

EMBED = 256
NH = 8
NL = 4
NP = 4
HD = 32
BS = 4
LQ = 5440
LV = 5440
NQ = BS * LQ
LVL_W = (64, 32, 16, 8)
LVL_START = (0, 4096, 5120, 5376)

NC, NS = 2, 16
NW = NC * NS
QW = NQ // NW
QB = 40
NBLK = QW // QB
WG = 4
NWAVE = QB // WG

MM_BT = 1360
PREP_QT = 272


def _proj_body(x_ref, w_ref, b_ref, o_ref):
    o_ref[...] = (jnp.dot(x_ref[...], w_ref[...],
                          preferred_element_type=jnp.float32) + b_ref[...])


def _proj_mask_body(x_ref, w_ref, b_ref, m_ref, o_ref):
    o_ref[...] = ((jnp.dot(x_ref[...], w_ref[...],
                           preferred_element_type=jnp.float32)
                   + b_ref[...]) * m_ref[...]).astype(jnp.bfloat16)


def _matmul_bias(x, w, b, mf=None):
    r = x.shape[0]
    grid = (r // MM_BT,)
    in_specs = [
        pl.BlockSpec((MM_BT, EMBED), lambda i: (i, 0)),
        pl.BlockSpec((EMBED, EMBED), lambda i: (0, 0)),
        pl.BlockSpec((1, EMBED), lambda i: (0, 0)),
    ]
    args = [x, w, b.reshape(1, EMBED)]
    body = _proj_body
    odtype = jnp.float32
    if mf is not None:
        in_specs.append(pl.BlockSpec((MM_BT, 1), lambda i: (i, 0)))
        args.append(mf)
        body = _proj_mask_body
        odtype = jnp.bfloat16
    return pl.pallas_call(
        body,
        grid=grid,
        in_specs=in_specs,
        out_specs=pl.BlockSpec((MM_BT, EMBED), lambda i: (i, 0)),
        out_shape=jax.ShapeDtypeStruct((r, EMBED), odtype),
    )(*args)


def _prep_body(q_ref, rx_ref, ry_ref, wox_ref, woy_ref, box_ref, boy_ref,
               wa_ref, ba_ref, i00_ref, i01_ref,
               w00_ref, w01_ref, w10_ref, w11_ref):
    b = pl.program_id(0)
    q = q_ref[0]
    rx = rx_ref[0]
    ry = ry_ref[0]

    offx = jnp.dot(q, wox_ref[...], preferred_element_type=jnp.float32) + box_ref[...]
    offy = jnp.dot(q, woy_ref[...], preferred_element_type=jnp.float32) + boy_ref[...]
    logits = jnp.dot(q, wa_ref[...], preferred_element_type=jnp.float32) + ba_ref[...]

    mrow = jnp.max(logits, axis=-1, keepdims=True)
    e = jnp.exp(logits - mrow)
    ri = lax.broadcasted_iota(jnp.int32, (128, 128), 0)
    ci = lax.broadcasted_iota(jnp.int32, (128, 128), 1)
    bm = (ri // 16 == ci // 16).astype(jnp.float32)
    esum = jnp.dot(e, bm, preferred_element_type=jnp.float32)
    aw = e / esum

    li = lax.broadcasted_iota(jnp.int32, (1, 128), 1)
    l_vec = (li // 4) % 4
    h_vec = li // 16
    wl = jnp.where(l_vec == 0, LVL_W[0],
                   jnp.where(l_vec == 1, LVL_W[1],
                             jnp.where(l_vec == 2, LVL_W[2], LVL_W[3])))
    lvs = jnp.where(l_vec == 0, LVL_START[0],
                    jnp.where(l_vec == 1, LVL_START[1],
                              jnp.where(l_vec == 2, LVL_START[2], LVL_START[3])))
    wlf = wl.astype(jnp.float32)

    x = jnp.clip(rx * wlf + offx - 0.5, -2.0, wlf + 1.0)
    y = jnp.clip(ry * wlf + offy - 0.5, -2.0, wlf + 1.0)
    x0f = jnp.floor(x)
    y0f = jnp.floor(y)
    fx1 = x - x0f
    fx0 = 1.0 - fx1
    fy1 = y - y0f
    fy0 = 1.0 - fy1
    x0 = x0f.astype(jnp.int32)
    y0 = y0f.astype(jnp.int32)
    x1 = x0 + 1
    y1 = y0 + 1
    vx0 = ((x0 >= 0) & (x0 < wl)).astype(jnp.float32)
    vx1 = ((x1 >= 0) & (x1 < wl)).astype(jnp.float32)
    vy0 = ((y0 >= 0) & (y0 < wl)).astype(jnp.float32)
    vy1 = ((y1 >= 0) & (y1 < wl)).astype(jnp.float32)
    cy0 = jnp.clip(y0, 0, wl - 1)
    cy1 = jnp.clip(y1, 0, wl - 1)

    ax0 = fx0 * vx0
    ax1 = fx1 * vx1
    sw = x0 >= 0
    pa = jnp.where(sw, ax0, ax1)
    pb = jnp.where(sw, ax1, 0.0)
    px = jnp.clip(x0, 0, wl - 1)
    by0 = aw * fy0 * vy0
    by1 = aw * fy1 * vy1

    base = (b * LV + lvs)
    i00_ref[...] = (base + cy0 * wl + px) * NH + h_vec
    i01_ref[...] = (base + cy1 * wl + px) * NH + h_vec
    w00_ref[...] = by0 * pa
    w01_ref[...] = by0 * pb
    w10_ref[...] = by1 * pa
    w11_ref[...] = by1 * pb


def _prep(query, rx128, ry128, wox, woy, box, boy, wa, ba):
    grid = (BS, LQ // PREP_QT)
    qt = PREP_QT
    in_specs = [
        pl.BlockSpec((1, qt, EMBED), lambda b, i: (b, i, 0)),
        pl.BlockSpec((1, qt, 128), lambda b, i: (b, i, 0)),
        pl.BlockSpec((1, qt, 128), lambda b, i: (b, i, 0)),
        pl.BlockSpec((EMBED, 128), lambda b, i: (0, 0)),
        pl.BlockSpec((EMBED, 128), lambda b, i: (0, 0)),
        pl.BlockSpec((1, 128), lambda b, i: (0, 0)),
        pl.BlockSpec((1, 128), lambda b, i: (0, 0)),
        pl.BlockSpec((EMBED, 128), lambda b, i: (0, 0)),
        pl.BlockSpec((1, 128), lambda b, i: (0, 0)),
    ]
    nqt = LQ // qt
    ospec = pl.BlockSpec((qt, 128), lambda b, i: (b * nqt + i, 0))
    out_specs = [ospec] * 6
    out_shape = ([jax.ShapeDtypeStruct((NQ, 128), jnp.int32)] * 2
                 + [jax.ShapeDtypeStruct((NQ, 128), jnp.float32)] * 4)
    return pl.pallas_call(
        _prep_body, grid=grid, in_specs=in_specs,
        out_specs=out_specs, out_shape=out_shape,
    )(query, rx128, ry128, wox, woy, box.reshape(1, 128), boy.reshape(1, 128),
      wa, ba.reshape(1, 128))


_GDN = lax.GatherDimensionNumbers(offset_dims=(), collapsed_slice_dims=(0,),
                                  start_index_map=(0,))


def _bcast_lane(vec, i):
    idx = jnp.full((16, 1), i, jnp.int32)
    return lax.gather(vec, idx, _GDN, (1,),
                      mode=lax.GatherScatterMode.PROMISE_IN_BOUNDS)


def _sc_body(t_hbm, i0_hbm, i1_hbm,
             w0_hbm, w1_hbm, w2_hbm, w3_hbm, out_hbm,
             i0_v, i1_v, w0_v, w1_v, w2_v, w3_v,
             g_v, out_v, sem, sem_stage, sem_out):
    wid = lax.axis_index("s") * NC + lax.axis_index("c")
    idx_hbms = [i0_hbm, i1_hbm]
    wt_hbms = [w0_hbm, w1_hbm, w2_hbm, w3_hbm]
    idx_vs = [i0_v, i1_v]
    wt_vs = [w0_v, w1_v, w2_v, w3_v]

    def out_wait():
        pltpu.make_async_copy(out_v.at[0], out_hbm.at[pl.ds(wid * QW, QB)],
                              sem_out).wait()

    def blk_body(blk, carry):
        base = wid * QW + blk * QB
        gbase = (wid * QW) // WG + blk * NWAVE
        sb = lax.rem(blk, 2)
        for s in range(2):
            pltpu.sync_copy(idx_hbms[s].at[pl.ds(gbase, NWAVE)], idx_vs[s])
        for c in range(4):
            pltpu.sync_copy(wt_hbms[c].at[pl.ds(base, QB)], wt_vs[c])

        @pl.when(blk >= 2)
        def _drain_out():
            out_wait()

        for s in range(2):
            pltpu.async_copy(t_hbm.at[idx_vs[s].at[0]], g_v.at[0, s], sem)

        def w_body(wv_i, carry2):
            slot = lax.rem(wv_i, 2)
            nslot = 1 - slot

            @pl.when(wv_i < NWAVE - 1)
            def _prefetch():
                for s in range(2):
                    pltpu.async_copy(t_hbm.at[idx_vs[s].at[wv_i + 1]],
                                     g_v.at[nslot, s], sem)

            for s in range(2):
                pltpu.make_async_copy(t_hbm.at[idx_vs[s].at[0]],
                                      g_v.at[slot, s], sem).wait()

            def qk_body(qk, carry3):
                qq = wv_i * WG + qk
                for h in range(NH):
                    acc0 = jnp.zeros((16,), jnp.float32)
                    acc1 = jnp.zeros((16,), jnp.float32)
                    for s in range(2):
                        wva = wt_vs[2 * s][qq, pl.ds(h * 16, 16)]
                        wvb = wt_vs[2 * s + 1][qq, pl.ds(h * 16, 16)]
                        for g4 in range(4):
                            accp = None
                            for k in range(4):
                                i = g4 * 4 + k
                                wpa = plsc.pack(*((_bcast_lane(wva, i),) * 2),
                                                format=plsc.PackFormat.INTERLEAVED)
                                wpb = plsc.pack(*((_bcast_lane(wvb, i),) * 2),
                                                format=plsc.PackFormat.INTERLEAVED)
                                ga = g_v[slot, s, qk * 128 + h * 16 + i,
                                         pl.ds(0, HD)]
                                gb = g_v[slot, s, qk * 128 + h * 16 + i,
                                         pl.ds(HD, HD)]
                                pa = ga * wpa
                                accp = pa if accp is None else accp + pa
                                accp = accp + gb * wpb
                            e, o = plsc.unpack(accp,
                                               format=plsc.PackFormat.INTERLEAVED)
                            acc0 = acc0 + e
                            acc1 = acc1 + o
                    out_v[sb, qq, pl.ds(h * HD, 16)] = acc0
                    out_v[sb, qq, pl.ds(h * HD + 16, 16)] = acc1
                return carry3

            lax.fori_loop(0, WG, qk_body, 0)
            return carry2

        lax.fori_loop(0, NWAVE, w_body, 0)
        pltpu.async_copy(out_v.at[sb], out_hbm.at[pl.ds(base, QB)], sem_out)
        return carry

    lax.fori_loop(0, NBLK, blk_body, 0)
    out_wait()
    out_wait()


def _sc_sample(t, idxs, wts):
    mesh = plsc.VectorSubcoreMesh(core_axis_name="c", subcore_axis_name="s")
    f = pl.kernel(
        _sc_body,
        out_type=jax.ShapeDtypeStruct((NQ, EMBED), jnp.float32),
        mesh=mesh,
        compiler_params=pltpu.CompilerParams(use_tc_tiling_on_sc=False,
                                             needs_layout_passes=False),
        scratch_types=(
            [pltpu.VMEM((NWAVE, WG * 128), jnp.int32)] * 2
            + [pltpu.VMEM((QB, 128), jnp.float32)] * 4
            + [
                pltpu.VMEM((2, 2, WG * 128, 2 * HD), jnp.bfloat16),
                pltpu.VMEM((2, QB, EMBED), jnp.float32),
                pltpu.SemaphoreType.DMA,
                pltpu.SemaphoreType.DMA,
                pltpu.SemaphoreType.DMA,
            ]
        ),
    )
    return f(t, *idxs, *wts)


_LANE_L = np.array([l for h in range(NH) for l in range(NL) for p in range(NP)])

_OUT_PERM = np.array([h * HD + (2 * j if j < 16 else 2 * (j - 16) + 1)
                      for h in range(NH) for j in range(HD)])


def kernel(query, reference_points, value, value_spatial_shapes, value_mask,
           W_value, b_value, W_off, b_off, W_attn, b_attn, W_out, b_out):
    mf = jnp.where(value_mask, 0.0, 1.0).reshape(NQ, 1)
    v2 = _matmul_bias(value.reshape(NQ, EMBED), W_value, b_value, mf)
    flat = v2.reshape(NQ * NH, HD)
    shifted = jnp.concatenate(
        [flat[NH:], jnp.zeros((NH, HD), jnp.bfloat16)], axis=0)
    t = jnp.concatenate([flat, shifted], axis=1)

    rx128 = reference_points[..., 0][:, :, _LANE_L]
    ry128 = reference_points[..., 1][:, :, _LANE_L]
    wox = W_off.reshape(EMBED, 128, 2)[:, :, 0]
    woy = W_off.reshape(EMBED, 128, 2)[:, :, 1]
    box = b_off.reshape(128, 2)[:, 0]
    boy = b_off.reshape(128, 2)[:, 1]
    outs = _prep(query, rx128, ry128, wox, woy, box, boy, W_attn, b_attn)

    gidx = [o.reshape(NQ // WG, WG * 128) for o in outs[:2]]
    out = _sc_sample(t, gidx, outs[2:])

    res = _matmul_bias(out, W_out[_OUT_PERM], b_out)
    return res.reshape(BS, LQ, EMBED)


# speedup vs baseline: 1.0139x; 1.0139x over previous
import functools
import numpy as np
import jax
import jax.numpy as jnp
from jax import lax
from jax.experimental import pallas as pl
from jax.experimental.pallas import tpu as pltpu
from jax.experimental.pallas import tpu_sc as plsc

EMBED = 256
NH = 8
NL = 4
NP = 4
HD = 32
BS = 4
LQ = 5440
LV = 5440
NQ = BS * LQ
LVL_W = (64, 32, 16, 8)
LVL_START = (0, 4096, 5120, 5376)

NC, NS = 2, 16
NW = NC * NS
QW = NQ // NW
QB = 40
NBLK = QW // QB
WG = 4
NWAVE = QB // WG

MM_BT = 1360
PREP_QT = 272


def _proj_body(x_ref, w_ref, b_ref, o_ref):
    o_ref[...] = (jnp.dot(x_ref[...], w_ref[...],
                          preferred_element_type=jnp.float32) + b_ref[...])


def _proj_mask_body(x_ref, w_ref, b_ref, m_ref, o_ref):
    o_ref[...] = ((jnp.dot(x_ref[...], w_ref[...],
                           preferred_element_type=jnp.float32)
                   + b_ref[...]) * m_ref[...]).astype(jnp.bfloat16)


def _matmul_bias(x, w, b, mf=None):
    r = x.shape[0]
    grid = (r // MM_BT,)
    in_specs = [
        pl.BlockSpec((MM_BT, EMBED), lambda i: (i, 0)),
        pl.BlockSpec((EMBED, EMBED), lambda i: (0, 0)),
        pl.BlockSpec((1, EMBED), lambda i: (0, 0)),
    ]
    args = [x, w, b.reshape(1, EMBED)]
    body = _proj_body
    odtype = jnp.float32
    if mf is not None:
        in_specs.append(pl.BlockSpec((MM_BT, 1), lambda i: (i, 0)))
        args.append(mf)
        body = _proj_mask_body
        odtype = jnp.bfloat16
    return pl.pallas_call(
        body,
        grid=grid,
        in_specs=in_specs,
        out_specs=pl.BlockSpec((MM_BT, EMBED), lambda i: (i, 0)),
        out_shape=jax.ShapeDtypeStruct((r, EMBED), odtype),
    )(*args)


def _prep_body(q_ref, rx_ref, ry_ref, wox_ref, woy_ref, box_ref, boy_ref,
               wa_ref, ba_ref, i00_ref, i01_ref,
               w00_ref, w01_ref, w10_ref, w11_ref):
    b = pl.program_id(0)
    q = q_ref[0]
    rx = rx_ref[0]
    ry = ry_ref[0]

    offx = jnp.dot(q, wox_ref[...], preferred_element_type=jnp.float32) + box_ref[...]
    offy = jnp.dot(q, woy_ref[...], preferred_element_type=jnp.float32) + boy_ref[...]
    logits = jnp.dot(q, wa_ref[...], preferred_element_type=jnp.float32) + ba_ref[...]

    mrow = jnp.max(logits, axis=-1, keepdims=True)
    e = jnp.exp(logits - mrow)
    ri = lax.broadcasted_iota(jnp.int32, (128, 128), 0)
    ci = lax.broadcasted_iota(jnp.int32, (128, 128), 1)
    bm = (ri // 16 == ci // 16).astype(jnp.float32)
    esum = jnp.dot(e, bm, preferred_element_type=jnp.float32)
    aw = e / esum

    li = lax.broadcasted_iota(jnp.int32, (1, 128), 1)
    l_vec = (li // 4) % 4
    h_vec = li // 16
    wl = jnp.where(l_vec == 0, LVL_W[0],
                   jnp.where(l_vec == 1, LVL_W[1],
                             jnp.where(l_vec == 2, LVL_W[2], LVL_W[3])))
    lvs = jnp.where(l_vec == 0, LVL_START[0],
                    jnp.where(l_vec == 1, LVL_START[1],
                              jnp.where(l_vec == 2, LVL_START[2], LVL_START[3])))
    wlf = wl.astype(jnp.float32)

    x = jnp.clip(rx * wlf + offx - 0.5, -2.0, wlf + 1.0)
    y = jnp.clip(ry * wlf + offy - 0.5, -2.0, wlf + 1.0)
    x0f = jnp.floor(x)
    y0f = jnp.floor(y)
    fx1 = x - x0f
    fx0 = 1.0 - fx1
    fy1 = y - y0f
    fy0 = 1.0 - fy1
    x0 = x0f.astype(jnp.int32)
    y0 = y0f.astype(jnp.int32)
    x1 = x0 + 1
    y1 = y0 + 1
    vx0 = ((x0 >= 0) & (x0 < wl)).astype(jnp.float32)
    vx1 = ((x1 >= 0) & (x1 < wl)).astype(jnp.float32)
    vy0 = ((y0 >= 0) & (y0 < wl)).astype(jnp.float32)
    vy1 = ((y1 >= 0) & (y1 < wl)).astype(jnp.float32)
    cy0 = jnp.clip(y0, 0, wl - 1)
    cy1 = jnp.clip(y1, 0, wl - 1)

    ax0 = fx0 * vx0
    ax1 = fx1 * vx1
    sw = x0 >= 0
    pa = jnp.where(sw, ax0, ax1)
    pb = jnp.where(sw, ax1, 0.0)
    px = jnp.clip(x0, 0, wl - 1)
    by0 = aw * fy0 * vy0
    by1 = aw * fy1 * vy1

    base = (b * LV + lvs)
    i00_ref[...] = (base + cy0 * wl + px) * NH + h_vec
    i01_ref[...] = (base + cy1 * wl + px) * NH + h_vec
    w00_ref[...] = by0 * pa
    w01_ref[...] = by0 * pb
    w10_ref[...] = by1 * pa
    w11_ref[...] = by1 * pb


def _prep(query, rx128, ry128, wox, woy, box, boy, wa, ba):
    grid = (BS, LQ // PREP_QT)
    qt = PREP_QT
    in_specs = [
        pl.BlockSpec((1, qt, EMBED), lambda b, i: (b, i, 0)),
        pl.BlockSpec((1, qt, 128), lambda b, i: (b, i, 0)),
        pl.BlockSpec((1, qt, 128), lambda b, i: (b, i, 0)),
        pl.BlockSpec((EMBED, 128), lambda b, i: (0, 0)),
        pl.BlockSpec((EMBED, 128), lambda b, i: (0, 0)),
        pl.BlockSpec((1, 128), lambda b, i: (0, 0)),
        pl.BlockSpec((1, 128), lambda b, i: (0, 0)),
        pl.BlockSpec((EMBED, 128), lambda b, i: (0, 0)),
        pl.BlockSpec((1, 128), lambda b, i: (0, 0)),
    ]
    nqt = LQ // qt
    ospec = pl.BlockSpec((qt, 128), lambda b, i: (b * nqt + i, 0))
    out_specs = [ospec] * 6
    out_shape = ([jax.ShapeDtypeStruct((NQ, 128), jnp.int32)] * 2
                 + [jax.ShapeDtypeStruct((NQ, 128), jnp.float32)] * 4)
    return pl.pallas_call(
        _prep_body, grid=grid, in_specs=in_specs,
        out_specs=out_specs, out_shape=out_shape,
    )(query, rx128, ry128, wox, woy, box.reshape(1, 128), boy.reshape(1, 128),
      wa, ba.reshape(1, 128))


_GDN = lax.GatherDimensionNumbers(offset_dims=(), collapsed_slice_dims=(0,),
                                  start_index_map=(0,))


def _bcast_lane(vec, i):
    idx = jnp.full((16, 1), i, jnp.int32)
    return lax.gather(vec, idx, _GDN, (1,),
                      mode=lax.GatherScatterMode.PROMISE_IN_BOUNDS)


def _sc_body(t_hbm, i0_hbm, i1_hbm,
             w0_hbm, w1_hbm, w2_hbm, w3_hbm, out_hbm,
             i0_v, i1_v, w0_v, w1_v, w2_v, w3_v,
             g_v, out_v, sem, sem_stage, sem_out):
    wid = lax.axis_index("s") * NC + lax.axis_index("c")
    idx_hbms = [i0_hbm, i1_hbm]
    wt_hbms = [w0_hbm, w1_hbm, w2_hbm, w3_hbm]
    idx_vs = [i0_v, i1_v]
    wt_vs = [w0_v, w1_v, w2_v, w3_v]

    def out_wait():
        pltpu.make_async_copy(out_v.at[0], out_hbm.at[pl.ds(wid * QW, QB)],
                              sem_out).wait()

    def blk_body(blk, carry):
        base = wid * QW + blk * QB
        gbase = (wid * QW) // WG + blk * NWAVE
        sb = lax.rem(blk, 2)
        for s in range(2):
            pltpu.async_copy(idx_hbms[s].at[pl.ds(gbase, NWAVE)], idx_vs[s],
                             sem_stage)
        for c in range(4):
            pltpu.async_copy(wt_hbms[c].at[pl.ds(base, QB)], wt_vs[c],
                             sem_stage)

        @pl.when(blk >= 2)
        def _drain_out():
            out_wait()

        for s in range(2):
            pltpu.make_async_copy(idx_hbms[s].at[pl.ds(gbase, NWAVE)],
                                  idx_vs[s], sem_stage).wait()
        for c in range(4):
            pltpu.make_async_copy(wt_hbms[c].at[pl.ds(base, QB)], wt_vs[c],
                                  sem_stage).wait()

        for s in range(2):
            pltpu.async_copy(t_hbm.at[idx_vs[s].at[0]], g_v.at[0, s], sem)

        def w_body(wv_i, carry2):
            slot = lax.rem(wv_i, 2)
            nslot = 1 - slot

            @pl.when(wv_i < NWAVE - 1)
            def _prefetch():
                for s in range(2):
                    pltpu.async_copy(t_hbm.at[idx_vs[s].at[wv_i + 1]],
                                     g_v.at[nslot, s], sem)

            for s in range(2):
                pltpu.make_async_copy(t_hbm.at[idx_vs[s].at[0]],
                                      g_v.at[slot, s], sem).wait()

            def qk_body(qk, carry3):
                qq = wv_i * WG + qk
                for h in range(NH):
                    acc0 = jnp.zeros((16,), jnp.float32)
                    acc1 = jnp.zeros((16,), jnp.float32)
                    for s in range(2):
                        wva = wt_vs[2 * s][qq, pl.ds(h * 16, 16)]
                        wvb = wt_vs[2 * s + 1][qq, pl.ds(h * 16, 16)]
                        for g4 in range(4):
                            accp = None
                            for k in range(4):
                                i = g4 * 4 + k
                                wpa = plsc.pack(*((_bcast_lane(wva, i),) * 2),
                                                format=plsc.PackFormat.INTERLEAVED)
                                wpb = plsc.pack(*((_bcast_lane(wvb, i),) * 2),
                                                format=plsc.PackFormat.INTERLEAVED)
                                ga = g_v[slot, s, qk * 128 + h * 16 + i,
                                         pl.ds(0, HD)]
                                gb = g_v[slot, s, qk * 128 + h * 16 + i,
                                         pl.ds(HD, HD)]
                                pa = ga * wpa
                                accp = pa if accp is None else accp + pa
                                accp = accp + gb * wpb
                            e, o = plsc.unpack(accp,
                                               format=plsc.PackFormat.INTERLEAVED)
                            acc0 = acc0 + e
                            acc1 = acc1 + o
                    out_v[sb, qq, pl.ds(h * HD, 16)] = acc0
                    out_v[sb, qq, pl.ds(h * HD + 16, 16)] = acc1
                return carry3

            lax.fori_loop(0, WG, qk_body, 0)
            return carry2

        lax.fori_loop(0, NWAVE, w_body, 0)
        pltpu.async_copy(out_v.at[sb], out_hbm.at[pl.ds(base, QB)], sem_out)
        return carry

    lax.fori_loop(0, NBLK, blk_body, 0)
    out_wait()
    out_wait()


def _sc_sample(t, idxs, wts):
    mesh = plsc.VectorSubcoreMesh(core_axis_name="c", subcore_axis_name="s")
    f = pl.kernel(
        _sc_body,
        out_type=jax.ShapeDtypeStruct((NQ, EMBED), jnp.float32),
        mesh=mesh,
        compiler_params=pltpu.CompilerParams(use_tc_tiling_on_sc=False,
                                             needs_layout_passes=False),
        scratch_types=(
            [pltpu.VMEM((NWAVE, WG * 128), jnp.int32)] * 2
            + [pltpu.VMEM((QB, 128), jnp.float32)] * 4
            + [
                pltpu.VMEM((2, 2, WG * 128, 2 * HD), jnp.bfloat16),
                pltpu.VMEM((2, QB, EMBED), jnp.float32),
                pltpu.SemaphoreType.DMA,
                pltpu.SemaphoreType.DMA,
                pltpu.SemaphoreType.DMA,
            ]
        ),
    )
    return f(t, *idxs, *wts)


_LANE_L = np.array([l for h in range(NH) for l in range(NL) for p in range(NP)])

_OUT_PERM = np.array([h * HD + (2 * j if j < 16 else 2 * (j - 16) + 1)
                      for h in range(NH) for j in range(HD)])


def kernel(query, reference_points, value, value_spatial_shapes, value_mask,
           W_value, b_value, W_off, b_off, W_attn, b_attn, W_out, b_out):
    mf = jnp.where(value_mask, 0.0, 1.0).reshape(NQ, 1)
    v2 = _matmul_bias(value.reshape(NQ, EMBED), W_value, b_value, mf)
    flat = v2.reshape(NQ * NH, HD)
    shifted = jnp.concatenate(
        [flat[NH:], jnp.zeros((NH, HD), jnp.bfloat16)], axis=0)
    t = jnp.concatenate([flat, shifted], axis=1)

    rx128 = reference_points[..., 0][:, :, _LANE_L]
    ry128 = reference_points[..., 1][:, :, _LANE_L]
    wox = W_off.reshape(EMBED, 128, 2)[:, :, 0]
    woy = W_off.reshape(EMBED, 128, 2)[:, :, 1]
    box = b_off.reshape(128, 2)[:, 0]
    boy = b_off.reshape(128, 2)[:, 1]
    outs = _prep(query, rx128, ry128, wox, woy, box, boy, W_attn, b_attn)

    gidx = [o.reshape(NQ // WG, WG * 128) for o in outs[:2]]
    out = _sc_sample(t, gidx, outs[2:])

    res = _matmul_bias(out, W_out[_OUT_PERM], b_out)
    return res.reshape(BS, LQ, EMBED)
